# baseline (speedup 1.0000x reference)
import functools

import jax
import jax.numpy as jnp
from jax import lax
from jax.experimental import pallas as pl
from jax.experimental.pallas import tpu as pltpu
from jax.experimental.pallas import tpu_sc as plsc

_BATCH = 4096
_SEQ = 200
_EMB = 64
_TOTAL = _BATCH * _SEQ
_NW = 32
_B_PER_W = _TOTAL // _NW
_CHUNK = 640
_NCHUNK = _B_PER_W // _CHUNK
_NBUF = 2
_KSUB = 8
_SUB = _CHUNK // _KSUB

_mesh = plsc.VectorSubcoreMesh(core_axis_name="c", subcore_axis_name="s")


@functools.partial(
    pl.kernel,
    mesh=_mesh,
    out_type=jax.ShapeDtypeStruct((_TOTAL, 128), jnp.float32),
    scratch_types=[
        pltpu.VMEM((_B_PER_W,), jnp.int32),
        pltpu.VMEM((_NBUF, _CHUNK, _EMB), jnp.float32),
        pltpu.SemaphoreType.DMA((_NBUF,)),
        pltpu.SemaphoreType.DMA((_NBUF,)),
    ],
    compiler_params=pltpu.CompilerParams(use_tc_tiling_on_sc=False),
)
def _gather_kernel(ids_hbm, table_hbm, out_hbm, idx_v, rows_v, gsem, osem):
    wid = lax.axis_index("s") * 2 + lax.axis_index("c")
    base = wid * _B_PER_W

    pltpu.sync_copy(ids_hbm.at[pl.ds(base, _B_PER_W)], idx_v)

    def gather_start(j, b):
        for k in range(_KSUB):
            idx = idx_v.at[pl.ds(j * _CHUNK + k * _SUB, _SUB)]
            dst = rows_v.at[b, pl.ds(k * _SUB, _SUB)]
            pltpu.async_copy(table_hbm.at[idx], dst, gsem.at[b])

    def gather_wait(j, b):
        for k in range(_KSUB):
            idx = idx_v.at[pl.ds(j * _CHUNK + k * _SUB, _SUB)]
            dst = rows_v.at[b, pl.ds(k * _SUB, _SUB)]
            pltpu.make_async_copy(table_hbm.at[idx], dst, gsem.at[b]).wait()

    def store_start(j, b):
        out = out_hbm.at[pl.ds(base + j * _CHUNK, _CHUNK), pl.ds(0, _EMB)]
        pltpu.async_copy(rows_v.at[b], out, osem.at[b])

    def store_wait(j, b):
        out = out_hbm.at[pl.ds(base + j * _CHUNK, _CHUNK), pl.ds(0, _EMB)]
        pltpu.make_async_copy(rows_v.at[b], out, osem.at[b]).wait()

    for b in range(_NBUF):
        gather_start(b, b)

    def body(t, carry):
        for b in range(_NBUF):
            j = t * _NBUF + b
            gather_wait(j, b)
            store_start(j, b)

            @pl.when(j < _NCHUNK - _NBUF)
            def _():
                store_wait(j, b)
                gather_start(j + _NBUF, b)

        return carry

    lax.fori_loop(0, _NCHUNK // _NBUF, body, 0)

    for b in range(_NBUF):
        store_wait(_NCHUNK - _NBUF + b, b)


def kernel(prompt_token_ids, embedding_table):
    ids = prompt_token_ids.reshape(_TOTAL)
    out = _gather_kernel(ids, embedding_table)
    return out[:, :_EMB].reshape(_BATCH, _SEQ, _EMB)


# speedup vs baseline: 1.0775x; 1.0775x over previous
import functools

import jax
import jax.numpy as jnp
from jax import lax
from jax.experimental import pallas as pl
from jax.experimental.pallas import tpu as pltpu
from jax.experimental.pallas import tpu_sc as plsc

_BATCH = 4096
_SEQ = 200
_EMB = 64
_TOTAL = _BATCH * _SEQ
_NW = 32
_B_PER_W = _TOTAL // _NW
_CHUNK = 640
_NCHUNK = _B_PER_W // _CHUNK
_NBUF = 2
_KSUB = 8
_SUB = _CHUNK // _KSUB

_mesh = plsc.VectorSubcoreMesh(core_axis_name="c", subcore_axis_name="s")


@functools.partial(
    pl.kernel,
    mesh=_mesh,
    out_type=jax.ShapeDtypeStruct((_TOTAL, 128), jnp.float32),
    scratch_types=[
        pltpu.VMEM((_B_PER_W,), jnp.int32),
        pltpu.VMEM((_NBUF, _CHUNK, _EMB), jnp.float32),
        pltpu.SemaphoreType.DMA((_NBUF,)),
        pltpu.SemaphoreType.DMA((_NBUF,)),
    ],
    compiler_params=pltpu.CompilerParams(use_tc_tiling_on_sc=False),
)
def _gather_kernel(ids_hbm, table_hbm, out_hbm, idx_v, rows_v, gsem, osem):
    wid = lax.axis_index("s") * 2 + lax.axis_index("c")
    base = wid * _B_PER_W

    pltpu.sync_copy(ids_hbm.at[pl.ds(base, _B_PER_W)], idx_v)

    def dbl(q, c):
        idx_v[pl.ds(16 * q, 16)] = lax.shift_left(
            idx_v[pl.ds(16 * q, 16)], 1
        )
        return c

    lax.fori_loop(0, _B_PER_W // 16, dbl, 0, unroll=8)

    def gather_start(j, b):
        for k in range(_KSUB):
            idx = idx_v.at[pl.ds(j * _CHUNK + k * _SUB, _SUB)]
            dst = rows_v.at[b, pl.ds(k * _SUB, _SUB)]
            pltpu.async_copy(table_hbm.at[idx], dst, gsem.at[b])

    def gather_wait(j, b):
        for k in range(_KSUB):
            idx = idx_v.at[pl.ds(j * _CHUNK + k * _SUB, _SUB)]
            dst = rows_v.at[b, pl.ds(k * _SUB, _SUB)]
            pltpu.make_async_copy(table_hbm.at[idx], dst, gsem.at[b]).wait()

    def store_start(j, b):
        out = out_hbm.at[pl.ds(base + j * _CHUNK, _CHUNK), pl.ds(0, _EMB)]
        pltpu.async_copy(rows_v.at[b], out, osem.at[b])

    def store_wait(j, b):
        out = out_hbm.at[pl.ds(base + j * _CHUNK, _CHUNK), pl.ds(0, _EMB)]
        pltpu.make_async_copy(rows_v.at[b], out, osem.at[b]).wait()

    for b in range(_NBUF):
        gather_start(b, b)

    def body(t, carry):
        for b in range(_NBUF):
            j = t * _NBUF + b
            gather_wait(j, b)
            store_start(j, b)

            @pl.when(j < _NCHUNK - _NBUF)
            def _():
                store_wait(j, b)
                gather_start(j + _NBUF, b)

        return carry

    lax.fori_loop(0, _NCHUNK // _NBUF, body, 0)

    for b in range(_NBUF):
        store_wait(_NCHUNK - _NBUF + b, b)


def kernel(prompt_token_ids, embedding_table):
    ids = prompt_token_ids.reshape(_TOTAL)
    tab = jnp.pad(embedding_table, ((0, 0), (0, 64))).reshape(2 * 1000000, _EMB)
    out = _gather_kernel(ids, tab)
    return out[:, :_EMB].reshape(_BATCH, _SEQ, _EMB)
